# baseline (speedup 1.0000x reference)
import jax
import jax.numpy as jnp
import numpy as np
from jax import lax
from jax.experimental import pallas as pl
from jax.experimental.pallas import tpu as pltpu
from jax.experimental.pallas import tpu_sc as plsc

N = 25000
D = 128
H = 64
E = 400000
HW = H // 2

NSUB = 16
CHUNK = 128
EC = 3200
EPAD = EC * CHUNK
CPT = EC // NSUB
IB = 8
NPAD = 25088
RPT = NPAD // NSUB

BN = 3136
GRID = NPAD // BN

_PERM = np.concatenate([np.arange(0, 32, 2), np.arange(1, 32, 2),
                        np.arange(32, 64, 2), np.arange(33, 64, 2)])


def _tc_pre_body(x0_ref, x1_ref, wrel0_ref, wroot0_ref, b0_ref,
                 wrel1_ref, wroot1_ref, b1_ref,
                 y0_ref, y1_ref, r0_ref, r1_ref):
    dn = (((1,), (1,)), ((), ()))
    x0 = x0_ref[...]
    x1 = x1_ref[...]
    y0_ref[...] = lax.dot_general(x0, wrel0_ref[...], dn,
                                  preferred_element_type=jnp.float32
                                  ).astype(jnp.bfloat16)
    y1_ref[...] = lax.dot_general(x1, wrel1_ref[...], dn,
                                  preferred_element_type=jnp.float32
                                  ).astype(jnp.bfloat16)
    r0_ref[...] = lax.dot_general(x0, wroot0_ref[...], dn,
                                  preferred_element_type=jnp.float32) + b0_ref[...]
    r1_ref[...] = lax.dot_general(x1, wroot1_ref[...], dn,
                                  preferred_element_type=jnp.float32) + b1_ref[...]


def _tc_pre(x0, x1, wrel0, wroot0, b0, wrel1, wroot1, b1):
    row_spec = pl.BlockSpec((BN, D), lambda i: (i, 0))
    w_spec = pl.BlockSpec((H, D), lambda i: (0, 0))
    b_spec = pl.BlockSpec((1, H), lambda i: (0, 0))
    out_spec = pl.BlockSpec((BN, H), lambda i: (i, 0))
    return pl.pallas_call(
        _tc_pre_body,
        grid=(GRID,),
        in_specs=[row_spec, row_spec, w_spec, w_spec, b_spec,
                  w_spec, w_spec, b_spec],
        out_specs=[out_spec] * 4,
        out_shape=[jax.ShapeDtypeStruct((NPAD, H), jnp.bfloat16),
                   jax.ShapeDtypeStruct((NPAD, H), jnp.bfloat16),
                   jax.ShapeDtypeStruct((NPAD, H), jnp.float32),
                   jax.ShapeDtypeStruct((NPAD, H), jnp.float32)],
    )(x0, x1, wrel0, wroot0, b0, wrel1, wroot1, b1)


def _unpack_chunk(r32, rf):
    mask = jnp.int32(-65536)

    @plsc.parallel_loop(0, CHUNK, unroll=8)
    def _(row):
        for k in (0, 16):
            w = r32[row, pl.ds(k, 16)]
            rf[row, pl.ds(2 * k, 16)] = plsc.bitcast(
                lax.shift_left(w, 16), jnp.float32)
            rf[row, pl.ds(2 * k + 16, 16)] = plsc.bitcast(
                w & mask, jnp.float32)


def _sc_body(y0, y1, r0, r1, src0, dst0, src1, dst1,
             o0, o1, idx_s, idx_d, r32a, r32b, rfa, rfb, acc,
             gsem0, gsem1, ssem0, ssem1):
    cid = lax.axis_index("c")
    sid = lax.axis_index("s")
    r32 = (r32a, r32b)
    rf = (rfa, rfb)
    gsem = (gsem0, gsem1)
    ssem = (ssem0, ssem1)

    def run(y, r, src, dst, out):
        base = sid * RPT
        pltpu.sync_copy(r.at[pl.ds(base, RPT)], acc.at[pl.ds(base, RPT)])
        cb = sid * CPT
        plsc.subcore_barrier()

        def outer(o, carry):
            ob = cb + o * IB
            pltpu.sync_copy(src.at[pl.ds(ob, IB)], idx_s)
            pltpu.sync_copy(dst.at[pl.ds(ob, IB)], idx_d)

            g = [None] * IB
            s = [None] * IB
            g[0] = pltpu.async_copy(y.at[idx_s.at[0]], r32[0], gsem[0])
            for j in range(IB):
                if j + 1 < IB:
                    g[j + 1] = pltpu.async_copy(
                        y.at[idx_s.at[j + 1]], r32[(j + 1) % 2],
                        gsem[(j + 1) % 2])
                g[j].wait()
                if j >= 2:
                    s[j - 2].wait()
                _unpack_chunk(r32[j % 2], rf[j % 2])
                s[j] = pltpu.async_copy(rf[j % 2], acc.at[idx_d.at[j]],
                                        ssem[j % 2], add=True)
            s[IB - 2].wait()
            s[IB - 1].wait()
            return carry

        lax.fori_loop(0, CPT // IB, outer, 0)
        plsc.subcore_barrier()
        pltpu.sync_copy(acc.at[pl.ds(base, RPT)], out.at[pl.ds(base, RPT)])

    @pl.when(cid == 0)
    def _():
        run(y0, r0, src0, dst0, o0)

    @pl.when(cid == 1)
    def _():
        run(y1, r1, src1, dst1, o1)


_sc_call = pl.kernel(
    _sc_body,
    out_type=[jax.ShapeDtypeStruct((NPAD, H), jnp.float32)] * 2,
    mesh=plsc.VectorSubcoreMesh(core_axis_name="c", subcore_axis_name="s"),
    scratch_types=[
        pltpu.VMEM((IB, CHUNK), jnp.int32),
        pltpu.VMEM((IB, CHUNK), jnp.int32),
        pltpu.VMEM((CHUNK, HW), jnp.int32),
        pltpu.VMEM((CHUNK, HW), jnp.int32),
        pltpu.VMEM((CHUNK, H), jnp.float32),
        pltpu.VMEM((CHUNK, H), jnp.float32),
        pltpu.VMEM_SHARED((NPAD, H), jnp.float32),
        pltpu.SemaphoreType.DMA,
        pltpu.SemaphoreType.DMA,
        pltpu.SemaphoreType.DMA,
        pltpu.SemaphoreType.DMA,
    ],
    compiler_params=pltpu.CompilerParams(use_tc_tiling_on_sc=False,
                                         needs_layout_passes=False),
)


def _tc_post_body(s0_ref, s1_ref, w0_ref, w1_ref, bfc_ref, out_ref):
    a0 = jnp.maximum(s0_ref[...], 0.0)
    a1 = jnp.maximum(s1_ref[...], 0.0)
    out_ref[...] = (jnp.sum(a0 * w0_ref[...], axis=1, keepdims=True)
                    + jnp.sum(a1 * w1_ref[...], axis=1, keepdims=True)
                    + bfc_ref[...])


def _tc_post(s0, s1, w0, w1, bfc):
    s_spec = pl.BlockSpec((BN, H), lambda i: (i, 0))
    w_spec = pl.BlockSpec((1, H), lambda i: (0, 0))
    return pl.pallas_call(
        _tc_post_body,
        grid=(GRID,),
        in_specs=[s_spec, s_spec, w_spec, w_spec,
                  pl.BlockSpec((1, 1), lambda i: (0, 0))],
        out_specs=pl.BlockSpec((BN, 1), lambda i: (i, 0)),
        out_shape=jax.ShapeDtypeStruct((NPAD, 1), jnp.float32),
    )(s0, s1, w0, w1, bfc)


def _prep_edges(edge_index):
    src = edge_index[0].astype(jnp.int32)
    dst = edge_index[1].astype(jnp.int32)
    pad = EPAD - E
    src = jnp.concatenate([src, jnp.zeros((pad,), jnp.int32)]).reshape(EC, CHUNK)
    dst = jnp.concatenate([dst, jnp.full((pad,), N, jnp.int32)]).reshape(EC, CHUNK)
    return src, dst


def kernel(x_v0, x_v1, edge_index_v0v1, edge_index_v1v0,
           W_rel0, b_rel0, W_root0, W_rel1, b_rel1, W_root1, W_fc, b_fc):
    src0, dst0 = _prep_edges(edge_index_v0v1)
    src1, dst1 = _prep_edges(edge_index_v1v0)

    perm = jnp.asarray(_PERM)
    y0, y1, r0, r1 = _tc_pre(x_v0, x_v1,
                             W_rel0, W_root0[perm], b_rel0[perm].reshape(1, H),
                             W_rel1, W_root1[perm], b_rel1[perm].reshape(1, H))

    y0p = lax.bitcast_convert_type(y0.reshape(NPAD, HW, 2), jnp.int32)
    y1p = lax.bitcast_convert_type(y1.reshape(NPAD, HW, 2), jnp.int32)

    s0, s1 = _sc_call(y0p, y1p, r0, r1, src0, dst0, src1, dst1)

    wfc0 = W_fc[0, :H][perm].reshape(1, H)
    wfc1 = W_fc[0, H:][perm].reshape(1, H)
    out = _tc_post(s0, s1, wfc0, wfc1, b_fc.reshape(1, 1))
    return out[:N]


# speedup vs baseline: 1.4871x; 1.4871x over previous
import jax
import jax.numpy as jnp
import numpy as np
from jax import lax
from jax.experimental import pallas as pl
from jax.experimental.pallas import tpu as pltpu
from jax.experimental.pallas import tpu_sc as plsc

N = 25000
D = 128
H = 64
E = 400000
HW = H // 2

NSUB = 16
CHUNK = 128
EC = 3200
EPAD = EC * CHUNK
CPT = EC // NSUB
IB = 8
NPAD = 25088
RPT = NPAD // NSUB

BN = 3136
GRID = NPAD // BN

_PERM = np.concatenate([np.arange(0, 16), np.arange(32, 48),
                        np.arange(16, 32), np.arange(48, 64)])


def _pack_pair(ya, yb):
    a = lax.bitcast_convert_type(ya.astype(jnp.bfloat16),
                                 jnp.uint16).astype(jnp.uint32)
    b = lax.bitcast_convert_type(yb.astype(jnp.bfloat16),
                                 jnp.uint16).astype(jnp.uint32)
    return lax.bitcast_convert_type(a | (b << jnp.uint32(16)), jnp.int32)


def _tc_pre_body(x0_ref, x1_ref, wr0a_ref, wr0b_ref, wr1a_ref, wr1b_ref,
                 wt0_ref, wt1_ref, b0_ref, b1_ref,
                 yp0_ref, yp1_ref, root_ref):
    dn = (((1,), (1,)), ((), ()))
    x0 = x0_ref[...]
    x1 = x1_ref[...]
    dot = lambda x, w: lax.dot_general(x, w[...], dn,
                                       preferred_element_type=jnp.float32)
    yp0_ref[...] = _pack_pair(dot(x0, wr0a_ref), dot(x0, wr0b_ref))
    yp1_ref[...] = _pack_pair(dot(x1, wr1a_ref), dot(x1, wr1b_ref))
    root_ref[:, :H] = dot(x0, wt0_ref) + b0_ref[...]
    root_ref[:, H:] = dot(x1, wt1_ref) + b1_ref[...]


def _tc_pre(x0, x1, wr0a, wr0b, wr1a, wr1b, wt0, wt1, b0, b1):
    row_spec = pl.BlockSpec((BN, D), lambda i: (i, 0))
    wh_spec = pl.BlockSpec((HW, D), lambda i: (0, 0))
    w_spec = pl.BlockSpec((H, D), lambda i: (0, 0))
    b_spec = pl.BlockSpec((1, H), lambda i: (0, 0))
    return pl.pallas_call(
        _tc_pre_body,
        grid=(GRID,),
        in_specs=[row_spec, row_spec, wh_spec, wh_spec, wh_spec, wh_spec,
                  w_spec, w_spec, b_spec, b_spec],
        out_specs=[pl.BlockSpec((BN, HW), lambda i: (i, 0)),
                   pl.BlockSpec((BN, HW), lambda i: (i, 0)),
                   pl.BlockSpec((BN, 2 * H), lambda i: (i, 0))],
        out_shape=[jax.ShapeDtypeStruct((NPAD, HW), jnp.int32),
                   jax.ShapeDtypeStruct((NPAD, HW), jnp.int32),
                   jax.ShapeDtypeStruct((NPAD, 2 * H), jnp.float32)],
    )(x0, x1, wr0a, wr0b, wr1a, wr1b, wt0, wt1, b0, b1)


def _unpack_chunk(r32, rf):
    mask = jnp.int32(-65536)

    @plsc.parallel_loop(0, CHUNK, unroll=8)
    def _(row):
        for k in (0, 16):
            w = r32[row, pl.ds(k, 16)]
            rf[row, pl.ds(2 * k, 16)] = plsc.bitcast(
                lax.shift_left(w, 16), jnp.float32)
            rf[row, pl.ds(2 * k + 16, 16)] = plsc.bitcast(
                w & mask, jnp.float32)


def _sc_body(y0, y1, root, src0, dst0, src1, dst1,
             sout, idx_s, idx_d, r32a, r32b, rfa, rfb, acc,
             gsem0, gsem1, ssem0, ssem1):
    cid = lax.axis_index("c")
    sid = lax.axis_index("s")
    r32 = (r32a, r32b)
    rf = (rfa, rfb)
    gsem = (gsem0, gsem1)
    ssem = (ssem0, ssem1)

    def run(y, col, src, dst):
        base = sid * RPT
        pltpu.sync_copy(root.at[pl.ds(base, RPT), pl.ds(col, H)],
                        acc.at[pl.ds(base, RPT)])
        cb = sid * CPT
        plsc.subcore_barrier()

        def outer(o, carry):
            ob = cb + o * IB
            pltpu.sync_copy(src.at[pl.ds(ob, IB)], idx_s)
            pltpu.sync_copy(dst.at[pl.ds(ob, IB)], idx_d)

            g = [None] * IB
            s = [None] * IB
            g[0] = pltpu.async_copy(y.at[idx_s.at[0]], r32[0], gsem[0])
            for j in range(IB):
                if j + 1 < IB:
                    g[j + 1] = pltpu.async_copy(
                        y.at[idx_s.at[j + 1]], r32[(j + 1) % 2],
                        gsem[(j + 1) % 2])
                g[j].wait()
                if j >= 2:
                    s[j - 2].wait()
                _unpack_chunk(r32[j % 2], rf[j % 2])
                s[j] = pltpu.async_copy(rf[j % 2], acc.at[idx_d.at[j]],
                                        ssem[j % 2], add=True)
            s[IB - 2].wait()
            s[IB - 1].wait()
            return carry

        lax.fori_loop(0, CPT // IB, outer, 0)
        plsc.subcore_barrier()
        pltpu.sync_copy(acc.at[pl.ds(base, RPT)],
                        sout.at[pl.ds(base, RPT), pl.ds(col, H)])

    @pl.when(cid == 0)
    def _():
        run(y0, 0, src0, dst0)

    @pl.when(cid == 1)
    def _():
        run(y1, H, src1, dst1)


_sc_call = pl.kernel(
    _sc_body,
    out_type=jax.ShapeDtypeStruct((NPAD, 2 * H), jnp.float32),
    mesh=plsc.VectorSubcoreMesh(core_axis_name="c", subcore_axis_name="s"),
    scratch_types=[
        pltpu.VMEM((IB, CHUNK), jnp.int32),
        pltpu.VMEM((IB, CHUNK), jnp.int32),
        pltpu.VMEM((CHUNK, HW), jnp.int32),
        pltpu.VMEM((CHUNK, HW), jnp.int32),
        pltpu.VMEM((CHUNK, H), jnp.float32),
        pltpu.VMEM((CHUNK, H), jnp.float32),
        pltpu.VMEM_SHARED((NPAD, H), jnp.float32),
        pltpu.SemaphoreType.DMA,
        pltpu.SemaphoreType.DMA,
        pltpu.SemaphoreType.DMA,
        pltpu.SemaphoreType.DMA,
    ],
    compiler_params=pltpu.CompilerParams(use_tc_tiling_on_sc=False,
                                         needs_layout_passes=False),
)


def _tc_post_body(s_ref, w_ref, bfc_ref, out_ref):
    a = jnp.maximum(s_ref[...], 0.0)
    out_ref[...] = (jnp.sum(a * w_ref[...], axis=1, keepdims=True)
                    + bfc_ref[...])


def _tc_post(s, w, bfc):
    return pl.pallas_call(
        _tc_post_body,
        grid=(GRID,),
        in_specs=[pl.BlockSpec((BN, 2 * H), lambda i: (i, 0)),
                  pl.BlockSpec((1, 2 * H), lambda i: (0, 0)),
                  pl.BlockSpec((1, 1), lambda i: (0, 0))],
        out_specs=pl.BlockSpec((BN, 1), lambda i: (i, 0)),
        out_shape=jax.ShapeDtypeStruct((N, 1), jnp.float32),
    )(s, w, bfc)


def _prep_edges(edge_index):
    src = edge_index[0].astype(jnp.int32)
    dst = edge_index[1].astype(jnp.int32)
    pad = EPAD - E
    src = jnp.concatenate([src, jnp.zeros((pad,), jnp.int32)]).reshape(EC, CHUNK)
    dst = jnp.concatenate([dst, jnp.full((pad,), N, jnp.int32)]).reshape(EC, CHUNK)
    return src, dst


def kernel(x_v0, x_v1, edge_index_v0v1, edge_index_v1v0,
           W_rel0, b_rel0, W_root0, W_rel1, b_rel1, W_root1, W_fc, b_fc):
    src0, dst0 = _prep_edges(edge_index_v0v1)
    src1, dst1 = _prep_edges(edge_index_v1v0)

    perm = jnp.asarray(_PERM)
    y0p, y1p, root = _tc_pre(
        x_v0, x_v1,
        W_rel0[:HW], W_rel0[HW:], W_rel1[:HW], W_rel1[HW:],
        W_root0[perm], W_root1[perm],
        b_rel0[perm].reshape(1, H), b_rel1[perm].reshape(1, H))

    s = _sc_call(y0p, y1p, root, src0, dst0, src1, dst1)

    wcomb = jnp.concatenate([W_fc[0, :H][perm],
                             W_fc[0, H:][perm]]).reshape(1, 2 * H)
    return _tc_post(s, wcomb, b_fc.reshape(1, 1))


# speedup vs baseline: 1.5115x; 1.0164x over previous
import jax
import jax.numpy as jnp
import numpy as np
from jax import lax
from jax.experimental import pallas as pl
from jax.experimental.pallas import tpu as pltpu
from jax.experimental.pallas import tpu_sc as plsc

N = 25000
D = 128
H = 64
E = 400000
HW = H // 2

NSUB = 16
CHUNK = 128
EC = 3200
EPAD = EC * CHUNK
CPT = EC // NSUB
IB = 8
NPAD = 25088
RPT = NPAD // NSUB

BN = 3136
GRID = NPAD // BN

_PERM = np.concatenate([np.arange(0, 16), np.arange(32, 48),
                        np.arange(16, 32), np.arange(48, 64)])


def _pack_pair(ya, yb):
    a = lax.bitcast_convert_type(ya.astype(jnp.bfloat16),
                                 jnp.uint16).astype(jnp.uint32)
    b = lax.bitcast_convert_type(yb.astype(jnp.bfloat16),
                                 jnp.uint16).astype(jnp.uint32)
    return lax.bitcast_convert_type(a | (b << jnp.uint32(16)), jnp.int32)


def _tc_pre_body(x0_ref, x1_ref, wr0a_ref, wr0b_ref, wr1a_ref, wr1b_ref,
                 wt0_ref, wt1_ref, b0_ref, b1_ref,
                 yp0_ref, yp1_ref, root_ref):
    dn = (((1,), (1,)), ((), ()))
    x0 = x0_ref[...]
    x1 = x1_ref[...]
    dot = lambda x, w: lax.dot_general(x, w[...], dn,
                                       preferred_element_type=jnp.float32)
    yp0_ref[...] = _pack_pair(dot(x0, wr0a_ref), dot(x0, wr0b_ref))
    yp1_ref[...] = _pack_pair(dot(x1, wr1a_ref), dot(x1, wr1b_ref))
    root_ref[:, :H] = dot(x0, wt0_ref) + b0_ref[...]
    root_ref[:, H:] = dot(x1, wt1_ref) + b1_ref[...]


def _tc_pre(x0, x1, wr0a, wr0b, wr1a, wr1b, wt0, wt1, b0, b1):
    row_spec = pl.BlockSpec((BN, D), lambda i: (i, 0))
    wh_spec = pl.BlockSpec((HW, D), lambda i: (0, 0))
    w_spec = pl.BlockSpec((H, D), lambda i: (0, 0))
    b_spec = pl.BlockSpec((1, H), lambda i: (0, 0))
    return pl.pallas_call(
        _tc_pre_body,
        grid=(GRID,),
        in_specs=[row_spec, row_spec, wh_spec, wh_spec, wh_spec, wh_spec,
                  w_spec, w_spec, b_spec, b_spec],
        out_specs=[pl.BlockSpec((BN, HW), lambda i: (i, 0)),
                   pl.BlockSpec((BN, HW), lambda i: (i, 0)),
                   pl.BlockSpec((BN, 2 * H), lambda i: (i, 0))],
        out_shape=[jax.ShapeDtypeStruct((NPAD, HW), jnp.int32),
                   jax.ShapeDtypeStruct((NPAD, HW), jnp.int32),
                   jax.ShapeDtypeStruct((NPAD, 2 * H), jnp.float32)],
    )(x0, x1, wr0a, wr0b, wr1a, wr1b, wt0, wt1, b0, b1)


def _unpack_chunk(r32, rf):
    mask = jnp.int32(-65536)

    @plsc.parallel_loop(0, CHUNK, unroll=16)
    def _(row):
        for k in (0, 16):
            w = r32[row, pl.ds(k, 16)]
            rf[row, pl.ds(2 * k, 16)] = plsc.bitcast(
                lax.shift_left(w, 16), jnp.float32)
            rf[row, pl.ds(2 * k + 16, 16)] = plsc.bitcast(
                w & mask, jnp.float32)


def _sc_body(y0, y1, root, e0, e1,
             sout, idx, r32a, r32b, rfa, rfb, acc,
             gsem0, gsem1, ssem0, ssem1):
    cid = lax.axis_index("c")
    sid = lax.axis_index("s")
    r32 = (r32a, r32b)
    rf = (rfa, rfb)
    gsem = (gsem0, gsem1)
    ssem = (ssem0, ssem1)

    def run(y, col, edges):
        base = sid * RPT
        pltpu.sync_copy(root.at[pl.ds(base, RPT), pl.ds(col, H)],
                        acc.at[pl.ds(base, RPT)])
        cb = sid * CPT
        plsc.subcore_barrier()

        def outer(o, carry):
            ob = cb + o * IB
            pltpu.sync_copy(edges.at[pl.ds(ob, IB)], idx)

            g = [None] * IB
            s = [None] * IB
            g[0] = pltpu.async_copy(y.at[idx.at[0, 0]], r32[0], gsem[0])
            for j in range(IB):
                if j + 1 < IB:
                    g[j + 1] = pltpu.async_copy(
                        y.at[idx.at[j + 1, 0]], r32[(j + 1) % 2],
                        gsem[(j + 1) % 2])
                g[j].wait()
                if j >= 2:
                    s[j - 2].wait()
                _unpack_chunk(r32[j % 2], rf[j % 2])
                s[j] = pltpu.async_copy(rf[j % 2], acc.at[idx.at[j, 1]],
                                        ssem[j % 2], add=True)
            s[IB - 2].wait()
            s[IB - 1].wait()
            return carry

        lax.fori_loop(0, CPT // IB, outer, 0)
        plsc.subcore_barrier()
        pltpu.sync_copy(acc.at[pl.ds(base, RPT)],
                        sout.at[pl.ds(base, RPT), pl.ds(col, H)])

    @pl.when(cid == 0)
    def _():
        run(y0, 0, e0)

    @pl.when(cid == 1)
    def _():
        run(y1, H, e1)


_sc_call = pl.kernel(
    _sc_body,
    out_type=jax.ShapeDtypeStruct((NPAD, 2 * H), jnp.float32),
    mesh=plsc.VectorSubcoreMesh(core_axis_name="c", subcore_axis_name="s"),
    scratch_types=[
        pltpu.VMEM((IB, 2, CHUNK), jnp.int32),
        pltpu.VMEM((CHUNK, HW), jnp.int32),
        pltpu.VMEM((CHUNK, HW), jnp.int32),
        pltpu.VMEM((CHUNK, H), jnp.float32),
        pltpu.VMEM((CHUNK, H), jnp.float32),
        pltpu.VMEM_SHARED((NPAD, H), jnp.float32),
        pltpu.SemaphoreType.DMA,
        pltpu.SemaphoreType.DMA,
        pltpu.SemaphoreType.DMA,
        pltpu.SemaphoreType.DMA,
    ],
    compiler_params=pltpu.CompilerParams(use_tc_tiling_on_sc=False,
                                         needs_layout_passes=False),
)


def _tc_post_body(s_ref, w_ref, bfc_ref, out_ref):
    a = jnp.maximum(s_ref[...], 0.0)
    out_ref[...] = (jnp.sum(a * w_ref[...], axis=1, keepdims=True)
                    + bfc_ref[...])


def _tc_post(s, w, bfc):
    return pl.pallas_call(
        _tc_post_body,
        grid=(GRID,),
        in_specs=[pl.BlockSpec((BN, 2 * H), lambda i: (i, 0)),
                  pl.BlockSpec((1, 2 * H), lambda i: (0, 0)),
                  pl.BlockSpec((1, 1), lambda i: (0, 0))],
        out_specs=pl.BlockSpec((BN, 1), lambda i: (i, 0)),
        out_shape=jax.ShapeDtypeStruct((N, 1), jnp.float32),
    )(s, w, bfc)


def _prep_edges(edge_index):
    src = edge_index[0].astype(jnp.int32)
    dst = edge_index[1].astype(jnp.int32)
    pad = EPAD - E
    src = jnp.concatenate([src, jnp.zeros((pad,), jnp.int32)]).reshape(EC, 1, CHUNK)
    dst = jnp.concatenate([dst, jnp.full((pad,), N, jnp.int32)]).reshape(EC, 1, CHUNK)
    return jnp.concatenate([src, dst], axis=1)


def kernel(x_v0, x_v1, edge_index_v0v1, edge_index_v1v0,
           W_rel0, b_rel0, W_root0, W_rel1, b_rel1, W_root1, W_fc, b_fc):
    e0 = _prep_edges(edge_index_v0v1)
    e1 = _prep_edges(edge_index_v1v0)

    perm = jnp.asarray(_PERM)
    y0p, y1p, root = _tc_pre(
        x_v0, x_v1,
        W_rel0[:HW], W_rel0[HW:], W_rel1[:HW], W_rel1[HW:],
        W_root0[perm], W_root1[perm],
        b_rel0[perm].reshape(1, H), b_rel1[perm].reshape(1, H))

    s = _sc_call(y0p, y1p, root, e0, e1)

    wcomb = jnp.concatenate([W_fc[0, :H][perm],
                             W_fc[0, H:][perm]]).reshape(1, 2 * H)
    return _tc_post(s, wcomb, b_fc.reshape(1, 1))
